# baseline (speedup 1.0000x reference)
import functools

import jax
import jax.numpy as jnp
from jax import lax
from jax.experimental import pallas as pl
from jax.experimental.pallas import tpu as pltpu
from jax.experimental.pallas import tpu_sc as plsc

VOCAB = 1000000
D_MODEL = 64
ROWS = 4096 * 200
K = 128
NUM_CHUNKS = ROWS // K
NBUF = 4

UCHUNK = 256
FULL_UCHUNKS = VOCAB // UCHUNK
UTAIL = VOCAB - FULL_UCHUNKS * UCHUNK


def _build_unpad(num_workers, nc):
    per_w = FULL_UCHUNKS // num_workers
    rem = FULL_UCHUNKS - per_w * num_workers
    pairs = per_w // 2
    mesh = plsc.VectorSubcoreMesh(core_axis_name="c", subcore_axis_name="s")

    @functools.partial(
        pl.kernel,
        mesh=mesh,
        out_type=jax.ShapeDtypeStruct((VOCAB // 2, 2 * D_MODEL), jnp.float32),
        scratch_types=[
            pltpu.VMEM((2, UCHUNK, D_MODEL), jnp.float32),
            pltpu.VMEM((2, UCHUNK // 2, 2 * D_MODEL), jnp.float32),
            pltpu.SemaphoreType.DMA((2,)),
            pltpu.SemaphoreType.DMA((2,)),
        ],
    )
    def unpad_kernel(t_hbm, dense_hbm, bufin, bufout, rsem, wsem):
        wid = lax.axis_index("s") * nc + lax.axis_index("c")

        def in_slice(chunk_id, n):
            base = pl.multiple_of(chunk_id * UCHUNK, UCHUNK)
            return t_hbm.at[pl.ds(base, n)]

        def out_slice(chunk_id, n):
            base = pl.multiple_of(chunk_id * (UCHUNK // 2), UCHUNK // 2)
            return dense_hbm.at[pl.ds(base, n)]

        def pack(s, nrows):
            def body(q, carry):
                for c in range(8):
                    val = bufin[s, 2 * q + (c // 4), pl.ds((c % 4) * 16, 16)]
                    bufout[s, q, pl.ds(c * 16, 16)] = val
                return carry

            lax.fori_loop(0, nrows, body, 0, unroll=False)

        pltpu.async_copy(in_slice(wid, UCHUNK), bufin.at[0], rsem.at[0])

        def outer(o, carry):
            for s in range(2):
                i = 2 * o + s
                cid = i * num_workers + wid
                pltpu.make_async_copy(in_slice(cid, UCHUNK), bufin.at[s],
                                      rsem.at[s]).wait()

                @pl.when(i + 1 < per_w)
                def _():
                    pltpu.async_copy(
                        in_slice((i + 1) * num_workers + wid, UCHUNK),
                        bufin.at[1 - s], rsem.at[1 - s])

                @pl.when(o >= 1)
                def _():
                    pltpu.make_async_copy(
                        bufout.at[s], out_slice(cid, UCHUNK // 2),
                        wsem.at[s]).wait()

                pack(s, UCHUNK // 2)
                pltpu.async_copy(bufout.at[s], out_slice(cid, UCHUNK // 2),
                                 wsem.at[s])
            return carry

        lax.fori_loop(0, pairs, outer, 0, unroll=False)
        for s in range(2):
            pltpu.make_async_copy(bufout.at[s], out_slice(wid, UCHUNK // 2),
                                  wsem.at[s]).wait()

        @pl.when(wid < rem)
        def _():
            cid = per_w * num_workers + wid
            pltpu.sync_copy(in_slice(cid, UCHUNK), bufin.at[0])
            pack(0, UCHUNK // 2)
            pltpu.sync_copy(bufout.at[0], out_slice(cid, UCHUNK // 2))

        @pl.when(wid == rem)
        def _():
            pltpu.sync_copy(t_hbm.at[pl.ds(FULL_UCHUNKS * UCHUNK, UTAIL)],
                            bufin.at[0, pl.ds(0, UTAIL)])
            pack(0, UTAIL // 2)
            pltpu.sync_copy(
                bufout.at[0, pl.ds(0, UTAIL // 2)],
                dense_hbm.at[pl.ds(FULL_UCHUNKS * (UCHUNK // 2), UTAIL // 2)])

    return unpad_kernel


def _build_gather(num_workers, nc):
    chunks_per_w = NUM_CHUNKS // num_workers
    pairs_per_w = chunks_per_w // 2
    mesh = plsc.VectorSubcoreMesh(core_axis_name="c", subcore_axis_name="s")

    @functools.partial(
        pl.kernel,
        mesh=mesh,
        out_type=jax.ShapeDtypeStruct((ROWS, D_MODEL), jnp.float32),
        compiler_params=pltpu.CompilerParams(use_tc_tiling_on_sc=False),
        scratch_types=[
            pltpu.VMEM((chunks_per_w, K), jnp.int32),
            pltpu.VMEM((NBUF, 2 * K, D_MODEL), jnp.float32),
            pltpu.SemaphoreType.DMA((NBUF,)),
            pltpu.SemaphoreType.DMA((NBUF,)),
        ],
    )
    def gather_kernel(idx_hbm, table_hbm, out_hbm, idx_v, rows_v, gsem, wsem):
        wid = lax.axis_index("s") * nc + lax.axis_index("c")
        cbase = wid * chunks_per_w
        pltpu.sync_copy(idx_hbm.at[pl.ds(cbase, chunks_per_w)], idx_v)

        def fire_pair(p, b):
            pltpu.async_copy(table_hbm.at[idx_v.at[2 * p]],
                             rows_v.at[b, pl.ds(0, K)], gsem.at[b])
            pltpu.async_copy(table_hbm.at[idx_v.at[2 * p + 1]],
                             rows_v.at[b, pl.ds(K, K)], gsem.at[b])

        for b in range(NBUF):
            fire_pair(b, b)

        def outer(i, carry):
            p0 = i * NBUF
            for b in range(NBUF):
                p = p0 + b
                pltpu.make_async_copy(out_hbm.at[pl.ds(0, 2 * K)],
                                      rows_v.at[b], gsem.at[b]).wait()
                dst = out_hbm.at[pl.ds((cbase + 2 * p) * K, 2 * K)]
                pltpu.async_copy(rows_v.at[b], dst, wsem.at[b])

                @pl.when(p + NBUF < pairs_per_w)
                def _():
                    pltpu.make_async_copy(rows_v.at[b], dst,
                                          wsem.at[b]).wait()
                    fire_pair(p + NBUF, b)
            return carry

        lax.fori_loop(0, pairs_per_w // NBUF, outer, 0, unroll=False)

        for b in range(NBUF):
            pltpu.make_async_copy(
                rows_v.at[b], out_hbm.at[pl.ds(cbase * K, 2 * K)],
                wsem.at[b]).wait()

    return gather_kernel


def kernel(inputs, table):
    info = plsc.get_sparse_core_info()
    nc = info.num_cores
    num_workers = nc * info.num_subcores
    idx = inputs.reshape(NUM_CHUNKS, K).astype(jnp.int32)
    dense = _build_unpad(num_workers, nc)(table)
    dense64 = dense.reshape(VOCAB, D_MODEL)
    out = _build_gather(num_workers, nc)(idx, dense64)
    return out.reshape(inputs.shape + (D_MODEL,))


# speedup vs baseline: 1.1590x; 1.1590x over previous
import functools

import jax
import jax.numpy as jnp
from jax import lax
from jax.experimental import pallas as pl
from jax.experimental.pallas import tpu as pltpu
from jax.experimental.pallas import tpu_sc as plsc

VOCAB = 1000000
D_MODEL = 64
ROWS = 4096 * 200
K = 128
NUM_CHUNKS = ROWS // K
PAIRS = NUM_CHUNKS // 2
NBUF = 4


def _build_gather(num_workers):
    chunks_per_w = NUM_CHUNKS // num_workers
    pairs_per_w = chunks_per_w // 2
    mesh = plsc.VectorSubcoreMesh(core_axis_name="c", subcore_axis_name="s")
    nc = 2

    @functools.partial(
        pl.kernel,
        mesh=mesh,
        out_type=jax.ShapeDtypeStruct((ROWS, D_MODEL), jnp.float32),
        compiler_params=pltpu.CompilerParams(use_tc_tiling_on_sc=False),
        scratch_types=[
            pltpu.VMEM((chunks_per_w, K), jnp.int32),
            pltpu.VMEM((NBUF, 2 * K, D_MODEL), jnp.float32),
            pltpu.SemaphoreType.DMA((NBUF,)),
            pltpu.SemaphoreType.DMA((NBUF,)),
        ],
    )
    def gather_kernel(idx_hbm, table_hbm, out_hbm, idx_v, rows_v, gsem, wsem):
        wid = lax.axis_index("s") * nc + lax.axis_index("c")
        cbase = wid * chunks_per_w
        pltpu.sync_copy(idx_hbm.at[pl.ds(cbase, chunks_per_w)], idx_v)

        def fire_pair(p, b):
            pltpu.async_copy(table_hbm.at[idx_v.at[2 * p]],
                             rows_v.at[b, pl.ds(0, K)], gsem.at[b])
            pltpu.async_copy(table_hbm.at[idx_v.at[2 * p + 1]],
                             rows_v.at[b, pl.ds(K, K)], gsem.at[b])

        for b in range(NBUF):
            fire_pair(b, b)

        def outer(i, carry):
            p0 = i * NBUF
            for b in range(NBUF):
                p = p0 + b
                pltpu.make_async_copy(out_hbm.at[pl.ds(0, 2 * K)],
                                      rows_v.at[b], gsem.at[b]).wait()
                dst = out_hbm.at[pl.ds((cbase + 2 * p) * K, 2 * K)]
                pltpu.async_copy(rows_v.at[b], dst, wsem.at[b])

                @pl.when(p + NBUF < pairs_per_w)
                def _():
                    pltpu.make_async_copy(rows_v.at[b], dst,
                                          wsem.at[b]).wait()
                    fire_pair(p + NBUF, b)
            return carry

        lax.fori_loop(0, pairs_per_w // NBUF, outer, 0, unroll=False)

        for b in range(NBUF):
            pltpu.make_async_copy(
                rows_v.at[b], out_hbm.at[pl.ds(cbase * K, 2 * K)],
                wsem.at[b]).wait()

    return gather_kernel


def kernel(inputs, table):
    info = plsc.get_sparse_core_info()
    num_workers = info.num_cores * info.num_subcores
    idx = inputs.reshape(NUM_CHUNKS, K).astype(jnp.int32)
    out = _build_gather(num_workers)(idx, table)
    return out.reshape(inputs.shape + (D_MODEL,))
